# baseline (speedup 1.0000x reference)
import functools

import jax
import jax.numpy as jnp
from jax import lax
from jax.experimental import pallas as pl
from jax.experimental.pallas import tpu as pltpu
from jax.experimental.pallas import tpu_sc as plsc

_MARGIN = 0.25
_GAMMA = 64.0
_B = 1024
_N = 100000
_H = 2000
_NBLK = _N // _H
_LOG2E = 1.4426950408889634
_LN2 = 0.6931471805599453
_C0 = _GAMMA * _MARGIN * _MARGIN * _LOG2E
_CS = 9.60897927029168
_NEG = -1e30

_SC_NC = 2
_SC_NS = 16
_SC_NW = _SC_NC * _SC_NS
_SC_BPW = _B // _SC_NW


def _sc_gather_body(feat_hbm, tgt_hbm, out_hbm, tgt_v, rows_v, vals_v, sem):
    wid = lax.axis_index("s") * _SC_NC + lax.axis_index("c")
    base = wid * _SC_BPW
    pltpu.sync_copy(tgt_hbm.at[pl.ds(base, _SC_BPW)], tgt_v)
    pltpu.async_copy(feat_hbm.at[tgt_v], rows_v, sem).wait()
    iota = lax.iota(jnp.int32, 16)
    for half in range(_SC_BPW // 16):
        acc = jnp.zeros((16,), jnp.float32)
        for l in range(16):
            j = 16 * half + l
            v = rows_v[j, pl.ds(base + 16 * half, 16)]
            acc = jnp.where(iota == l, v, acc)
        vals_v[pl.ds(16 * half, 16)] = acc
    pltpu.sync_copy(vals_v, out_hbm.at[pl.ds(base, _SC_BPW)])


@functools.lru_cache(maxsize=1)
def _sc_gather():
    return functools.partial(
        pl.kernel,
        mesh=plsc.VectorSubcoreMesh(core_axis_name="c", subcore_axis_name="s"),
        out_type=jax.ShapeDtypeStruct((_B,), jnp.float32),
        scratch_types=[
            pltpu.VMEM((_SC_BPW,), jnp.int32),
            pltpu.VMEM((_SC_BPW, _B), jnp.float32),
            pltpu.VMEM((_SC_BPW,), jnp.float32),
            pltpu.SemaphoreType.DMA,
        ],
    )(_sc_gather_body)


def _dense_body(tgt_ref, feat_ref, m_ref, s_ref, mx_ref, iota_ref):
    c = pl.program_id(0)

    @pl.when(c == 0)
    def _init():
        m_ref[...] = jnp.full((1, _B), _C0, jnp.float32)
        s_ref[...] = jnp.zeros((1, _B), jnp.float32)
        mx_ref[...] = jnp.full((1, _B), _NEG, jnp.float32)
        iota_ref[...] = lax.broadcasted_iota(jnp.int32, (_H, _B), 0)

    x = feat_ref[...]
    iota = iota_ref[...]
    tsh = tgt_ref[...] - c * _H
    xm = jnp.where(iota == tsh, _NEG, x)
    mx_new = jnp.maximum(mx_ref[...], jnp.max(xm, axis=0, keepdims=True))
    yb = jnp.maximum(mx_new, _MARGIN) * _CS
    m_new = yb * yb
    y = jnp.maximum(xm, -_MARGIN) * _CS
    s_ref[...] = s_ref[...] * jnp.exp2(m_ref[...] - m_new) + jnp.sum(
        jnp.exp2(y * y - m_new), axis=0, keepdims=True
    )
    m_ref[...] = m_new
    mx_ref[...] = mx_new


def _dense_call(tgt, feat_t, interpret=False):
    return pl.pallas_call(
        _dense_body,
        grid=(_NBLK,),
        in_specs=[
            pl.BlockSpec((1, _B), lambda c: (0, 0)),
            pl.BlockSpec((_H, _B), lambda c: (c, 0)),
        ],
        out_specs=[
            pl.BlockSpec((1, _B), lambda c: (0, 0)),
            pl.BlockSpec((1, _B), lambda c: (0, 0)),
        ],
        out_shape=[
            jax.ShapeDtypeStruct((1, _B), jnp.float32),
            jax.ShapeDtypeStruct((1, _B), jnp.float32),
        ],
        scratch_shapes=[
            pltpu.VMEM((1, _B), jnp.float32),
            pltpu.VMEM((_H, _B), jnp.int32),
        ],
        interpret=interpret,
    )(tgt, feat_t)


def _combine_body(m_ref, s_ref, tv_ref, out_ref):
    tv = tv_ref[...]
    fin_t = jnp.maximum(1.0 + _MARGIN - tv, 0.0) * ((tv - (1.0 - _MARGIN)) * _GAMMA)
    h_t = fin_t * _LOG2E + _C0
    m = m_ref[...]
    s = s_ref[...]
    big = jnp.maximum(m, h_t)
    tot = s * jnp.exp2(m - big) + jnp.exp2(h_t - big)
    lse = (big - _C0 + jnp.log2(tot)) * _LN2
    out_ref[...] = jnp.mean(lse - fin_t).reshape(1, 1)


def _combine_call(m, s, tv, interpret=False):
    return pl.pallas_call(
        _combine_body,
        out_shape=jax.ShapeDtypeStruct((1, 1), jnp.float32),
        interpret=interpret,
    )(m, s, tv)


@jax.jit
def kernel(feat, target):
    tgt1d = target.astype(jnp.int32)
    feat_t = feat.T
    tv = _sc_gather()(feat_t, tgt1d).reshape(1, _B)
    m, s = _dense_call(tgt1d.reshape(1, _B), feat_t)
    out = _combine_call(m, s, tv)
    return out[0, 0]


# speedup vs baseline: 1.0026x; 1.0026x over previous
import functools

import jax
import jax.numpy as jnp
from jax import lax
from jax.experimental import pallas as pl
from jax.experimental.pallas import tpu as pltpu
from jax.experimental.pallas import tpu_sc as plsc

_MARGIN = 0.25
_GAMMA = 64.0
_B = 1024
_N = 100000
_H = 2000
_NBLK = _N // _H
_LOG2E = 1.4426950408889634
_LN2 = 0.6931471805599453
_C0 = _GAMMA * _MARGIN * _MARGIN * _LOG2E
_CS = 9.60897927029168
_NEG = -1e30

_SC_NC = 2
_SC_NS = 16
_SC_NW = _SC_NC * _SC_NS
_SC_BPW = _B // _SC_NW


def _sc_gather_body(feat_hbm, tgt_hbm, out_hbm, tgt_v, rows_v, vals_v, sem):
    wid = lax.axis_index("s") * _SC_NC + lax.axis_index("c")
    base = wid * _SC_BPW
    pltpu.sync_copy(tgt_hbm.at[pl.ds(base, _SC_BPW)], tgt_v)
    pltpu.async_copy(feat_hbm.at[tgt_v], rows_v, sem).wait()
    iota = lax.iota(jnp.int32, 16)
    for half in range(_SC_BPW // 16):
        acc = jnp.zeros((16,), jnp.float32)
        for l in range(16):
            j = 16 * half + l
            v = rows_v[j, pl.ds(base + 16 * half, 16)]
            acc = jnp.where(iota == l, v, acc)
        vals_v[pl.ds(16 * half, 16)] = acc
    pltpu.sync_copy(vals_v, out_hbm.at[pl.ds(base, _SC_BPW)])


@functools.lru_cache(maxsize=1)
def _sc_gather():
    return functools.partial(
        pl.kernel,
        mesh=plsc.VectorSubcoreMesh(core_axis_name="c", subcore_axis_name="s"),
        out_type=jax.ShapeDtypeStruct((_B,), jnp.float32),
        scratch_types=[
            pltpu.VMEM((_SC_BPW,), jnp.int32),
            pltpu.VMEM((_SC_BPW, _B), jnp.float32),
            pltpu.VMEM((_SC_BPW,), jnp.float32),
            pltpu.SemaphoreType.DMA,
        ],
    )(_sc_gather_body)


def _dense_body(tgt_ref, feat_ref, m_ref, s_ref, mx_ref):
    c = pl.program_id(0)

    @pl.when(c == 0)
    def _init():
        m_ref[...] = jnp.full((1, _B), _C0, jnp.float32)
        s_ref[...] = jnp.zeros((1, _B), jnp.float32)
        mx_ref[...] = jnp.full((1, _B), _NEG, jnp.float32)

    x = feat_ref[...]
    iota = lax.broadcasted_iota(jnp.int32, (_H, _B), 0)
    tsh = tgt_ref[...] - c * _H
    xm = jnp.where(iota == tsh, _NEG, x)
    mx_new = jnp.maximum(mx_ref[...], jnp.max(xm, axis=0, keepdims=True))
    yb = jnp.maximum(mx_new, _MARGIN) * _CS
    m_new = yb * yb
    y = jnp.maximum(xm, -_MARGIN) * _CS
    s_ref[...] = s_ref[...] * jnp.exp2(m_ref[...] - m_new) + jnp.sum(
        jnp.exp2(y * y - m_new), axis=0, keepdims=True
    )
    m_ref[...] = m_new
    mx_ref[...] = mx_new


def _dense_call(tgt, feat_t, interpret=False):
    return pl.pallas_call(
        _dense_body,
        grid=(_NBLK,),
        in_specs=[
            pl.BlockSpec((1, _B), lambda c: (0, 0)),
            pl.BlockSpec((_H, _B), lambda c: (c, 0)),
        ],
        out_specs=[
            pl.BlockSpec((1, _B), lambda c: (0, 0)),
            pl.BlockSpec((1, _B), lambda c: (0, 0)),
        ],
        out_shape=[
            jax.ShapeDtypeStruct((1, _B), jnp.float32),
            jax.ShapeDtypeStruct((1, _B), jnp.float32),
        ],
        scratch_shapes=[pltpu.VMEM((1, _B), jnp.float32)],
        interpret=interpret,
    )(tgt, feat_t)


def _combine_body(m_ref, s_ref, tv_ref, out_ref):
    tv = tv_ref[...]
    fin_t = jnp.maximum(1.0 + _MARGIN - tv, 0.0) * ((tv - (1.0 - _MARGIN)) * _GAMMA)
    h_t = fin_t * _LOG2E + _C0
    m = m_ref[...]
    s = s_ref[...]
    big = jnp.maximum(m, h_t)
    tot = s * jnp.exp2(m - big) + jnp.exp2(h_t - big)
    lse = (big - _C0 + jnp.log2(tot)) * _LN2
    out_ref[...] = jnp.mean(lse - fin_t).reshape(1, 1)


def _combine_call(m, s, tv, interpret=False):
    return pl.pallas_call(
        _combine_body,
        out_shape=jax.ShapeDtypeStruct((1, 1), jnp.float32),
        interpret=interpret,
    )(m, s, tv)


@jax.jit
def kernel(feat, target):
    tgt1d = target.astype(jnp.int32)
    feat_t = feat.T
    tv = _sc_gather()(feat_t, tgt1d).reshape(1, _B)
    m, s = _dense_call(tgt1d.reshape(1, _B), feat_t)
    out = _combine_call(m, s, tv)
    return out[0, 0]
